# baseline (speedup 1.0000x reference)
import functools

import jax
import jax.numpy as jnp
from jax.experimental import pallas as pl
from jax.experimental.pallas import tpu as pltpu


def _mof_body(x_ref, wg_ref, wm_ref, b_ref, o_ref, *, G, HPG):
    xb = x_ref[...]

    logits = jax.lax.dot_general(
        xb, wg_ref[...],
        dimension_numbers=(((1,), (1,)), ((), ())),
        preferred_element_type=jnp.float32)
    m1 = jnp.max(logits, axis=-1, keepdims=True)
    z = jnp.sum(jnp.exp(logits - m1), axis=-1, keepdims=True)
    v1 = 1.0 / z
    gi = jax.lax.broadcasted_iota(jnp.int32, logits.shape, 1)
    i1 = jnp.argmax(logits, axis=-1)
    oh1 = gi == i1[:, None]
    masked = jnp.where(oh1, -1e30, logits)
    i2 = jnp.argmax(masked, axis=-1)
    m2 = jnp.max(masked, axis=-1, keepdims=True)
    v2 = jnp.exp(m2 - m1) * v1
    oh2 = gi == i2[:, None]

    w1 = jnp.where(oh1, v1, 0.0)
    w2 = jnp.where(oh2, v2, 0.0)
    o1 = oh1.astype(jnp.float32)
    o2 = oh2.astype(jnp.float32)
    wp = 1.0 - o1 - o2

    sel0 = w1[:, 0:1] * xb[:, 0:HPG]
    sel1 = w2[:, 0:1] * xb[:, 0:HPG]
    for g in range(1, G):
        xg = xb[:, g * HPG:(g + 1) * HPG]
        sel0 = sel0 + w1[:, g:g + 1] * xg
        sel1 = sel1 + w2[:, g:g + 1] * xg
    flat = jnp.concatenate([sel0, sel1], axis=1)

    y = jax.lax.dot_general(
        flat, wm_ref[...],
        dimension_numbers=(((1,), (1,)), ((), ())),
        preferred_element_type=jnp.float32) + b_ref[...]
    y0 = y[:, 0:HPG]
    y1 = y[:, HPG:2 * HPG]

    for g in range(G):
        xg = xb[:, g * HPG:(g + 1) * HPG]
        og = xg * wp[:, g:g + 1] + y0 * o1[:, g:g + 1] + y1 * o2[:, g:g + 1]
        o_ref[:, g * HPG:(g + 1) * HPG] = og


def kernel(x, W_gate, W_model, b_model):
    b, l, h = x.shape
    G = W_gate.shape[0]
    HPG = h // G
    N = b * l
    T = 512
    while N % T:
        T //= 2

    xf = x.reshape(N, h)
    bm = b_model.reshape(1, -1)

    out = pl.pallas_call(
        functools.partial(_mof_body, G=G, HPG=HPG),
        grid=(N // T,),
        in_specs=[
            pl.BlockSpec((T, h), lambda i: (i, 0)),
            pl.BlockSpec((G, h), lambda i: (0, 0)),
            pl.BlockSpec(W_model.shape, lambda i: (0, 0)),
            pl.BlockSpec(bm.shape, lambda i: (0, 0)),
        ],
        out_specs=pl.BlockSpec((T, h), lambda i: (i, 0)),
        out_shape=jax.ShapeDtypeStruct((N, h), x.dtype),
        compiler_params=pltpu.CompilerParams(
            dimension_semantics=("parallel",)),
    )(xf, W_gate, W_model, bm)
    return out.reshape(b, l, h)


# speedup vs baseline: 2.8125x; 2.8125x over previous
import functools

import jax
import jax.numpy as jnp
from jax.experimental import pallas as pl
from jax.experimental.pallas import tpu as pltpu


def _mof_body(x_ref, wg_ref, wm_ref, b_ref, o_ref, *, G, HPG):
    xb = x_ref[...]
    T = xb.shape[0]

    logits = jax.lax.dot_general(
        xb, wg_ref[...],
        dimension_numbers=(((1,), (1,)), ((), ())),
        preferred_element_type=jnp.float32)
    m1 = jnp.max(logits, axis=-1)
    z = jnp.sum(jnp.exp(logits - m1[:, None]), axis=-1)
    v1 = 1.0 / z
    i1 = jnp.argmax(logits, axis=-1)
    gi = jax.lax.broadcasted_iota(jnp.int32, logits.shape, 1)
    l2 = jnp.where(gi == i1[:, None], -jnp.inf, logits)
    i2 = jnp.argmax(l2, axis=-1)
    v2 = jnp.exp(jnp.max(l2, axis=-1) - m1) * v1

    sel0 = xb[:, 0:HPG]
    sel1 = xb[:, 0:HPG]
    for g in range(1, G):
        xg = xb[:, g * HPG:(g + 1) * HPG]
        sel0 = jnp.where((i1 == g)[:, None], xg, sel0)
        sel1 = jnp.where((i2 == g)[:, None], xg, sel1)
    flat = jnp.concatenate([sel0 * v1[:, None], sel1 * v2[:, None]], axis=1)

    y = jax.lax.dot_general(
        flat, wm_ref[...],
        dimension_numbers=(((1,), (1,)), ((), ())),
        preferred_element_type=jnp.float32) + b_ref[...]
    y0 = y[:, 0:HPG]
    y1 = y[:, HPG:2 * HPG]

    for g in range(G):
        xg = xb[:, g * HPG:(g + 1) * HPG]
        og = jnp.where((i1 == g)[:, None], y0, xg)
        og = jnp.where((i2 == g)[:, None], y1, og)
        o_ref[:, g * HPG:(g + 1) * HPG] = og


def kernel(x, W_gate, W_model, b_model):
    b, l, h = x.shape
    G = W_gate.shape[0]
    HPG = h // G
    N = b * l
    T = 1024
    while N % T:
        T //= 2

    xf = x.reshape(N, h)
    bm = b_model.reshape(1, -1)

    out = pl.pallas_call(
        functools.partial(_mof_body, G=G, HPG=HPG),
        grid=(N // T,),
        in_specs=[
            pl.BlockSpec((T, h), lambda i: (i, 0)),
            pl.BlockSpec((G, h), lambda i: (0, 0)),
            pl.BlockSpec(W_model.shape, lambda i: (0, 0)),
            pl.BlockSpec(bm.shape, lambda i: (0, 0)),
        ],
        out_specs=pl.BlockSpec((T, h), lambda i: (i, 0)),
        out_shape=jax.ShapeDtypeStruct((N, h), x.dtype),
        compiler_params=pltpu.CompilerParams(
            dimension_semantics=("parallel",)),
    )(xf, W_gate, W_model, bm)
    return out.reshape(b, l, h)
